# Initial kernel scaffold; baseline (speedup 1.0000x reference)
import functools

import jax
import jax.numpy as jnp
from jax import lax
from jax.experimental import pallas as pl
from jax.experimental.pallas import tpu as pltpu
from jax.experimental.pallas import tpu_sc as plsc

NC = 2
NS = 16
CHUNK = 128


def _pre_body(x_ref, ws_ref, wt_ref, bs_ref, bv_ref, s_ref, y_ref):
  xb = x_ref[...]
  s_ref[...] = jnp.dot(xb, ws_ref[...], preferred_element_type=jnp.float32) + bs_ref[...]
  y_ref[...] = jnp.dot(xb, wt_ref[...], preferred_element_type=jnp.float32) + bv_ref[...]


def _pre(x, W_S, W_T, b_S, b_vec, block_rows):
  n, d = x.shape
  grid = (n // block_rows,)
  out = jax.ShapeDtypeStruct((n, d), jnp.float32)
  return pl.pallas_call(
      _pre_body,
      grid=grid,
      in_specs=[
          pl.BlockSpec((block_rows, d), lambda i: (i, 0)),
          pl.BlockSpec((d, d), lambda i: (0, 0)),
          pl.BlockSpec((d, d), lambda i: (0, 0)),
          pl.BlockSpec((1, d), lambda i: (0, 0)),
          pl.BlockSpec((1, d), lambda i: (0, 0)),
      ],
      out_specs=[
          pl.BlockSpec((block_rows, d), lambda i: (i, 0)),
          pl.BlockSpec((block_rows, d), lambda i: (i, 0)),
      ],
      out_shape=[out, out],
      compiler_params=pltpu.CompilerParams(
          dimension_semantics=("parallel",)),
  )(x, W_S, W_T, b_S, b_vec)


def _sc_agg_body(n_nodes, e_per_w, y_hbm, row_hbm, col_hbm, tf_hbm,
                 zacc_hbm, zt_hbm, acc_out, tagg_out,
                 acc_sp, tagg_sp, col_v, row_v, rows_v, tf_v,
                 col_r, row_r, rows_r, tf_r):
  c = lax.axis_index("c")
  s = lax.axis_index("s")
  wid = c * NS + s
  npt = n_nodes // NS
  nbase = s * npt

  pltpu.sync_copy(zacc_hbm.at[pl.ds(nbase, npt), :], acc_sp.at[pl.ds(nbase, npt), :])
  pltpu.sync_copy(zt_hbm.at[pl.ds(nbase, npt), :], tagg_sp.at[pl.ds(nbase, npt), :])
  plsc.subcore_barrier()

  base = wid * e_per_w
  nchunks = e_per_w // CHUNK
  rem = e_per_w - nchunks * CHUNK

  @pl.loop(0, nchunks)
  def _chunk(i):
    off = base + i * CHUNK
    pltpu.sync_copy(col_hbm.at[pl.ds(off, CHUNK)], col_v)
    pltpu.sync_copy(row_hbm.at[pl.ds(off, CHUNK)], row_v)
    pltpu.sync_copy(tf_hbm.at[pl.ds(off, CHUNK), :], tf_v)
    pltpu.sync_copy(y_hbm.at[col_v], rows_v)
    pltpu.sync_copy(rows_v, acc_sp.at[row_v], add=True)
    pltpu.sync_copy(tf_v, tagg_sp.at[row_v], add=True)

  if rem:
    off = base + nchunks * CHUNK
    pltpu.sync_copy(col_hbm.at[pl.ds(off, rem)], col_r)
    pltpu.sync_copy(row_hbm.at[pl.ds(off, rem)], row_r)
    pltpu.sync_copy(tf_hbm.at[pl.ds(off, rem), :], tf_r)
    pltpu.sync_copy(y_hbm.at[col_r], rows_r)
    pltpu.sync_copy(rows_r, acc_sp.at[row_r], add=True)
    pltpu.sync_copy(tf_r, tagg_sp.at[row_r], add=True)

  plsc.subcore_barrier()

  pltpu.sync_copy(acc_sp.at[pl.ds(nbase, npt), :], acc_out.at[c, pl.ds(nbase, npt), :])
  pltpu.sync_copy(tagg_sp.at[pl.ds(nbase, npt), :], tagg_out.at[c, pl.ds(nbase, npt), :])


def _sc_agg(y, row, col, tf, zacc, zt):
  n, d = y.shape
  (e,) = row.shape
  dt = tf.shape[1]
  e_per_w = e // (NC * NS)
  rem = e_per_w - (e_per_w // CHUNK) * CHUNK
  mesh = plsc.VectorSubcoreMesh(core_axis_name="c", subcore_axis_name="s")
  kern = pl.kernel(
      functools.partial(_sc_agg_body, n, e_per_w),
      out_type=[
          jax.ShapeDtypeStruct((NC, n, d), jnp.float32),
          jax.ShapeDtypeStruct((NC, n, dt), jnp.float32),
      ],
      mesh=mesh,
      scratch_types=[
          pltpu.MemoryRef((n, d), jnp.float32, memory_space=pltpu.VMEM_SHARED),
          pltpu.MemoryRef((n, dt), jnp.float32, memory_space=pltpu.VMEM_SHARED),
          pltpu.MemoryRef((CHUNK,), jnp.int32, memory_space=pltpu.VMEM),
          pltpu.MemoryRef((CHUNK,), jnp.int32, memory_space=pltpu.VMEM),
          pltpu.MemoryRef((CHUNK, d), jnp.float32, memory_space=pltpu.VMEM),
          pltpu.MemoryRef((CHUNK, dt), jnp.float32, memory_space=pltpu.VMEM),
          pltpu.MemoryRef((max(rem, 8),), jnp.int32, memory_space=pltpu.VMEM),
          pltpu.MemoryRef((max(rem, 8),), jnp.int32, memory_space=pltpu.VMEM),
          pltpu.MemoryRef((max(rem, 8), d), jnp.float32, memory_space=pltpu.VMEM),
          pltpu.MemoryRef((max(rem, 8), dt), jnp.float32, memory_space=pltpu.VMEM),
      ],
  )
  return kern(y, row, col, tf, zacc, zt)


def _post_body(s_ref, acc_ref, tagg_ref, wt_ref, o_ref):
  agg = acc_ref[0] + acc_ref[1]
  tg = tagg_ref[0] + tagg_ref[1]
  msg = agg + jnp.dot(tg, wt_ref[...], preferred_element_type=jnp.float32)
  o_ref[...] = jnp.maximum(s_ref[...] + msg, 0.0)


def _post(s, acc, tagg, W_tmp, block_rows):
  n, d = s.shape
  dt = W_tmp.shape[0]
  grid = (n // block_rows,)
  return pl.pallas_call(
      _post_body,
      grid=grid,
      in_specs=[
          pl.BlockSpec((block_rows, d), lambda i: (i, 0)),
          pl.BlockSpec((NC, block_rows, d), lambda i: (0, i, 0)),
          pl.BlockSpec((NC, block_rows, dt), lambda i: (0, i, 0)),
          pl.BlockSpec((dt, d), lambda i: (0, 0)),
      ],
      out_specs=pl.BlockSpec((block_rows, d), lambda i: (i, 0)),
      out_shape=jax.ShapeDtypeStruct((n, d), jnp.float32),
      compiler_params=pltpu.CompilerParams(
          dimension_semantics=("parallel",)),
  )(s, acc, tagg, W_tmp)


def kernel(x, edge_index, temporal_features, W_S, b_S, W_T, b_T, W_tmp, b_tmp):
  n, d = x.shape
  row = edge_index[0].astype(jnp.int32)
  col = edge_index[1].astype(jnp.int32)
  tf = temporal_features
  b_s2 = b_S.reshape(1, d).astype(jnp.float32)
  b_vec = (b_T + b_tmp).reshape(1, d).astype(jnp.float32)

  s_feat, y = _pre(x, W_S, W_T, b_s2, b_vec, block_rows=1000)

  zacc = jnp.zeros((n, d), jnp.float32)
  zt = jnp.zeros((n, tf.shape[1]), jnp.float32)
  acc, tagg = _sc_agg(y, row, col, tf, zacc, zt)

  return _post(s_feat, acc, tagg, W_tmp, block_rows=1000)


# speedup vs baseline: 2.8438x; 2.8438x over previous
import functools

import jax
import jax.numpy as jnp
from jax import lax
from jax.experimental import pallas as pl
from jax.experimental.pallas import tpu as pltpu
from jax.experimental.pallas import tpu_sc as plsc

NC = 2
NS = 16
CHUNK = 128


def _pre_body(x_ref, ws_ref, wt_ref, bs_ref, bv_ref, s_ref, y_ref):
  xb = x_ref[...]
  s_ref[...] = jnp.dot(xb, ws_ref[...], preferred_element_type=jnp.float32) + bs_ref[...]
  y_ref[...] = jnp.dot(xb, wt_ref[...], preferred_element_type=jnp.float32) + bv_ref[...]


def _pre(x, W_S, W_T, b_S, b_vec, block_rows):
  n, d = x.shape
  grid = (n // block_rows,)
  out = jax.ShapeDtypeStruct((n, d), jnp.float32)
  return pl.pallas_call(
      _pre_body,
      grid=grid,
      in_specs=[
          pl.BlockSpec((block_rows, d), lambda i: (i, 0)),
          pl.BlockSpec((d, d), lambda i: (0, 0)),
          pl.BlockSpec((d, d), lambda i: (0, 0)),
          pl.BlockSpec((1, d), lambda i: (0, 0)),
          pl.BlockSpec((1, d), lambda i: (0, 0)),
      ],
      out_specs=[
          pl.BlockSpec((block_rows, d), lambda i: (i, 0)),
          pl.BlockSpec((block_rows, d), lambda i: (i, 0)),
      ],
      out_shape=[out, out],
      compiler_params=pltpu.CompilerParams(
          dimension_semantics=("parallel",)),
  )(x, W_S, W_T, b_S, b_vec)


def _tenc_body(tf_ref, wt_ref, o_ref):
  o_ref[...] = jnp.dot(tf_ref[...], wt_ref[...], preferred_element_type=jnp.float32)


def _tenc(tf, W_tmp, block_rows):
  e, dt = tf.shape
  d = W_tmp.shape[1]
  grid = (e // block_rows,)
  return pl.pallas_call(
      _tenc_body,
      grid=grid,
      in_specs=[
          pl.BlockSpec((block_rows, dt), lambda i: (i, 0)),
          pl.BlockSpec((dt, d), lambda i: (0, 0)),
      ],
      out_specs=pl.BlockSpec((block_rows, d), lambda i: (i, 0)),
      out_shape=jax.ShapeDtypeStruct((e, d), jnp.float32),
      compiler_params=pltpu.CompilerParams(
          dimension_semantics=("parallel",)),
  )(tf, W_tmp)


def _sc_agg_body(n_pad, n_edges, y_hbm, row_hbm, col_hbm, tenc_hbm,
                 zacc_hbm, ids_hbm, acc_out,
                 acc_sp, ids_v, col_v, row_v, rows_v, tenc_v):
  c = lax.axis_index("c")
  s = lax.axis_index("s")
  wid = c * NS + s
  nw = NC * NS

  nzc = n_pad // CHUNK
  my_nz = (nzc - s + NS - 1) // NS

  @pl.loop(0, my_nz)
  def _zero(k):
    off = (s + k * NS) * CHUNK
    pltpu.sync_copy(ids_hbm.at[pl.ds(off, CHUNK)], ids_v)
    pltpu.sync_copy(zacc_hbm.at[pl.ds(off, CHUNK), :], rows_v)
    pltpu.sync_copy(rows_v, acc_sp.at[ids_v])

  plsc.subcore_barrier()

  nec = n_edges // CHUNK
  my_ne = (nec - wid + nw - 1) // nw

  @pl.loop(0, my_ne)
  def _edge(j):
    off = (wid + j * nw) * CHUNK
    pltpu.sync_copy(col_hbm.at[pl.ds(off, CHUNK)], col_v)
    pltpu.sync_copy(row_hbm.at[pl.ds(off, CHUNK)], row_v)
    pltpu.sync_copy(tenc_hbm.at[pl.ds(off, CHUNK), :], tenc_v)
    pltpu.sync_copy(y_hbm.at[col_v], rows_v)
    pltpu.sync_copy(rows_v, acc_sp.at[row_v], add=True)
    pltpu.sync_copy(tenc_v, acc_sp.at[row_v], add=True)

  plsc.subcore_barrier()

  @pl.loop(0, my_nz)
  def _wout(k):
    off = (s + k * NS) * CHUNK
    pltpu.sync_copy(ids_hbm.at[pl.ds(off, CHUNK)], ids_v)
    pltpu.sync_copy(acc_sp.at[ids_v], rows_v)
    pltpu.sync_copy(rows_v, acc_out.at[c, pl.ds(off, CHUNK), :])


def _sc_agg(y, row, col, tenc, zacc, ids):
  n_pad, d = zacc.shape
  (e,) = row.shape
  assert e % CHUNK == 0 and n_pad % CHUNK == 0
  mesh = plsc.VectorSubcoreMesh(core_axis_name="c", subcore_axis_name="s")
  kern = pl.kernel(
      functools.partial(_sc_agg_body, n_pad, e),
      out_type=jax.ShapeDtypeStruct((NC, n_pad, d), jnp.float32),
      mesh=mesh,
      scratch_types=[
          pltpu.VMEM_SHARED((n_pad, d), jnp.float32),
          pltpu.VMEM((CHUNK,), jnp.int32),
          pltpu.VMEM((CHUNK,), jnp.int32),
          pltpu.VMEM((CHUNK,), jnp.int32),
          pltpu.VMEM((CHUNK, d), jnp.float32),
          pltpu.VMEM((CHUNK, d), jnp.float32),
      ],
  )
  return kern(y, row, col, tenc, zacc, ids)


def _post_body(s_ref, acc_ref, o_ref):
  agg = acc_ref[0] + acc_ref[1]
  o_ref[...] = jnp.maximum(s_ref[...] + agg, 0.0)


def _post(s, acc, block_rows):
  n, d = s.shape
  grid = (n // block_rows,)
  return pl.pallas_call(
      _post_body,
      grid=grid,
      in_specs=[
          pl.BlockSpec((block_rows, d), lambda i: (i, 0)),
          pl.BlockSpec((NC, block_rows, d), lambda i: (0, i, 0)),
      ],
      out_specs=pl.BlockSpec((block_rows, d), lambda i: (i, 0)),
      out_shape=jax.ShapeDtypeStruct((n, d), jnp.float32),
      compiler_params=pltpu.CompilerParams(
          dimension_semantics=("parallel",)),
  )(s, acc)


def kernel(x, edge_index, temporal_features, W_S, b_S, W_T, b_T, W_tmp, b_tmp):
  n, d = x.shape
  row = edge_index[0].astype(jnp.int32)
  col = edge_index[1].astype(jnp.int32)
  tf = temporal_features
  b_s2 = b_S.reshape(1, d).astype(jnp.float32)
  b_vec = (b_T + b_tmp).reshape(1, d).astype(jnp.float32)

  s_feat, y = _pre(x, W_S, W_T, b_s2, b_vec, block_rows=1000)
  tenc = _tenc(tf, W_tmp, block_rows=2000)

  n_pad = ((n + CHUNK - 1) // CHUNK) * CHUNK
  zacc = jnp.zeros((n_pad, d), jnp.float32)
  ids = jnp.arange(n_pad, dtype=jnp.int32)
  acc = _sc_agg(y, row, col, tenc, zacc, ids)

  return _post(s_feat, acc, block_rows=1000)


# speedup vs baseline: 4.1423x; 1.4566x over previous
import functools

import jax
import jax.numpy as jnp
from jax import lax
from jax.experimental import pallas as pl
from jax.experimental.pallas import tpu as pltpu
from jax.experimental.pallas import tpu_sc as plsc

NC = 2
NS = 16
CHUNK = 128


def _pre_body(x_ref, ws_ref, wt_ref, bs_ref, bv_ref, s_ref, y_ref):
  xb = x_ref[...]
  s_ref[...] = jnp.dot(xb, ws_ref[...], preferred_element_type=jnp.float32) + bs_ref[...]
  y_ref[...] = jnp.dot(xb, wt_ref[...], preferred_element_type=jnp.float32) + bv_ref[...]


def _pre(x, W_S, W_T, b_S, b_vec, block_rows):
  n, d = x.shape
  grid = (n // block_rows,)
  out = jax.ShapeDtypeStruct((n, d), jnp.float32)
  return pl.pallas_call(
      _pre_body,
      grid=grid,
      in_specs=[
          pl.BlockSpec((block_rows, d), lambda i: (i, 0)),
          pl.BlockSpec((d, d), lambda i: (0, 0)),
          pl.BlockSpec((d, d), lambda i: (0, 0)),
          pl.BlockSpec((1, d), lambda i: (0, 0)),
          pl.BlockSpec((1, d), lambda i: (0, 0)),
      ],
      out_specs=[
          pl.BlockSpec((block_rows, d), lambda i: (i, 0)),
          pl.BlockSpec((block_rows, d), lambda i: (i, 0)),
      ],
      out_shape=[out, out],
      compiler_params=pltpu.CompilerParams(
          dimension_semantics=("parallel",)),
  )(x, W_S, W_T, b_S, b_vec)


def _sc_agg_body(n_pad, n_edges, y_hbm, row_hbm, col_hbm, tf_hbm,
                 zacc_hbm, zt_hbm, ids_hbm, acc_out, tagg_out,
                 acc_sp, tagg_sp, ids_v, col_v, row_v, rows_v, tf_v):
  c = lax.axis_index("c")
  s = lax.axis_index("s")
  wid = c * NS + s
  nw = NC * NS

  nzc = n_pad // CHUNK
  my_nz = (nzc - s + NS - 1) // NS

  @pl.loop(0, my_nz)
  def _zero(k):
    off = (s + k * NS) * CHUNK
    pltpu.sync_copy(ids_hbm.at[pl.ds(off, CHUNK)], ids_v)
    pltpu.sync_copy(zacc_hbm.at[pl.ds(off, CHUNK), :], rows_v)
    pltpu.sync_copy(zt_hbm.at[pl.ds(off, CHUNK), :], tf_v)
    pltpu.sync_copy(rows_v, acc_sp.at[ids_v])
    pltpu.sync_copy(tf_v, tagg_sp.at[ids_v])

  plsc.subcore_barrier()

  nec = n_edges // CHUNK
  my_ne = (nec - wid + nw - 1) // nw

  @pl.loop(0, my_ne)
  def _edge(j):
    off = (wid + j * nw) * CHUNK
    pltpu.sync_copy(col_hbm.at[pl.ds(off, CHUNK)], col_v)
    pltpu.sync_copy(row_hbm.at[pl.ds(off, CHUNK)], row_v)
    pltpu.sync_copy(tf_hbm.at[pl.ds(off, CHUNK), :], tf_v)
    pltpu.sync_copy(y_hbm.at[col_v], rows_v)
    pltpu.sync_copy(rows_v, acc_sp.at[row_v], add=True)
    pltpu.sync_copy(tf_v, tagg_sp.at[row_v], add=True)

  plsc.subcore_barrier()

  @pl.loop(0, my_nz)
  def _wout(k):
    off = (s + k * NS) * CHUNK
    pltpu.sync_copy(ids_hbm.at[pl.ds(off, CHUNK)], ids_v)
    pltpu.sync_copy(acc_sp.at[ids_v], rows_v)
    pltpu.sync_copy(tagg_sp.at[ids_v], tf_v)
    pltpu.sync_copy(rows_v, acc_out.at[c, pl.ds(off, CHUNK), :])
    pltpu.sync_copy(tf_v, tagg_out.at[c, pl.ds(off, CHUNK), :])


def _sc_agg(y, row, col, tf, zacc, zt, ids):
  n_pad, d = zacc.shape
  (e,) = row.shape
  dt = tf.shape[1]
  assert e % CHUNK == 0 and n_pad % CHUNK == 0
  mesh = plsc.VectorSubcoreMesh(core_axis_name="c", subcore_axis_name="s")
  kern = pl.kernel(
      functools.partial(_sc_agg_body, n_pad, e),
      out_type=[
          jax.ShapeDtypeStruct((NC, n_pad, d), jnp.float32),
          jax.ShapeDtypeStruct((NC, n_pad, dt), jnp.float32),
      ],
      mesh=mesh,
      compiler_params=pltpu.CompilerParams(use_tc_tiling_on_sc=False),
      scratch_types=[
          pltpu.VMEM_SHARED((n_pad, d), jnp.float32),
          pltpu.VMEM_SHARED((n_pad, dt), jnp.float32),
          pltpu.VMEM((CHUNK,), jnp.int32),
          pltpu.VMEM((CHUNK,), jnp.int32),
          pltpu.VMEM((CHUNK,), jnp.int32),
          pltpu.VMEM((CHUNK, d), jnp.float32),
          pltpu.VMEM((CHUNK, dt), jnp.float32),
      ],
  )
  return kern(y, row, col, tf, zacc, zt, ids)


def _post_body(s_ref, acc_ref, tagg_ref, wt_ref, o_ref):
  agg = acc_ref[0] + acc_ref[1]
  tg = tagg_ref[0] + tagg_ref[1]
  msg = agg + jnp.dot(tg, wt_ref[...], preferred_element_type=jnp.float32)
  o_ref[...] = jnp.maximum(s_ref[...] + msg, 0.0)


def _post(s, acc, tagg, W_tmp, block_rows):
  n, d = s.shape
  dt = W_tmp.shape[0]
  grid = (n // block_rows,)
  return pl.pallas_call(
      _post_body,
      grid=grid,
      in_specs=[
          pl.BlockSpec((block_rows, d), lambda i: (i, 0)),
          pl.BlockSpec((NC, block_rows, d), lambda i: (0, i, 0)),
          pl.BlockSpec((NC, block_rows, dt), lambda i: (0, i, 0)),
          pl.BlockSpec((dt, d), lambda i: (0, 0)),
      ],
      out_specs=pl.BlockSpec((block_rows, d), lambda i: (i, 0)),
      out_shape=jax.ShapeDtypeStruct((n, d), jnp.float32),
      compiler_params=pltpu.CompilerParams(
          dimension_semantics=("parallel",)),
  )(s, acc, tagg, W_tmp)


def kernel(x, edge_index, temporal_features, W_S, b_S, W_T, b_T, W_tmp, b_tmp):
  n, d = x.shape
  row = edge_index[0].astype(jnp.int32)
  col = edge_index[1].astype(jnp.int32)
  tf = temporal_features
  b_s2 = b_S.reshape(1, d).astype(jnp.float32)
  b_vec = (b_T + b_tmp).reshape(1, d).astype(jnp.float32)

  s_feat, y = _pre(x, W_S, W_T, b_s2, b_vec, block_rows=1000)

  n_pad = ((n + CHUNK - 1) // CHUNK) * CHUNK
  zacc = jnp.zeros((n_pad, d), jnp.float32)
  zt = jnp.zeros((n_pad, tf.shape[1]), jnp.float32)
  ids = jnp.arange(n_pad, dtype=jnp.int32)
  acc, tagg = _sc_agg(y, row, col, tf, zacc, zt, ids)

  return _post(s_feat, acc, tagg, W_tmp, block_rows=1000)
